# baseline (speedup 1.0000x reference)
import functools

import jax
import jax.numpy as jnp
import numpy as np
from jax import lax
from jax.experimental import pallas as pl
from jax.experimental.pallas import tpu as pltpu
from jax.experimental.pallas import tpu_sc as plsc

K_LEVEL = 16
HK = K_LEVEL // 2
MBS, NCOBJ = 4, 8
BS = MBS * NCOBJ
NUM_TOP = 128
BAG = 64
D = 512
X = BAG * HK
NPAIR = 2 * BS
ROWS_PER_PAIR = NUM_TOP
TOT_IDX = NPAIR * NUM_TOP * HK
G_ROWS = NPAIR * ROWS_PER_PAIR

NW = 32
PAIRS_PER_W = NPAIR // NW
IDX_PER_W = PAIRS_PER_W * NUM_TOP * HK
CHUNK_ROWS = 16
TOPS_PER_CHUNK = CHUNK_ROWS // HK
NCHUNK = IDX_PER_W // CHUNK_ROWS
LANES = 16
GROUPS = D // LANES


def _sc_gather_segsum(table_hbm, idx_hbm, out_hbm, idx_v, buf0, buf1, buf2,
                      buf3, buf4, buf5, buf6, buf7, outb0, outb1, sg0, sg1,
                      sg2, sg3, sg4, sg5, sg6, sg7, so0, so1):
    nc = 2
    wid = lax.axis_index("s") * nc + lax.axis_index("c")

    pltpu.sync_copy(idx_hbm.at[pl.ds(wid * IDX_PER_W, IDX_PER_W)], idx_v)

    def start_gather(c, buf, sem):
        src = table_hbm.at[idx_v.at[pl.ds(c * CHUNK_ROWS, CHUNK_ROWS)]]
        pltpu.async_copy(src, buf, sem)

    def wait_gather(buf, sem):
        pltpu.make_async_copy(table_hbm.at[pl.ds(0, CHUNK_ROWS)], buf,
                              sem).wait()

    def accumulate(buf, outb):
        @plsc.parallel_loop(0, TOPS_PER_CHUNK * GROUPS, unroll=8)
        def _(it):
            t = it // GROUPS
            g = it % GROUPS
            r0 = t * HK
            sl = pl.ds(g * LANES, LANES)
            v01 = buf[r0, sl] + buf[r0 + 1, sl]
            v23 = buf[r0 + 2, sl] + buf[r0 + 3, sl]
            v45 = buf[r0 + 4, sl] + buf[r0 + 5, sl]
            v67 = buf[r0 + 6, sl] + buf[r0 + 7, sl]
            outb[t, sl] = (v01 + v23) + (v45 + v67)

    out_base = wid * (PAIRS_PER_W * ROWS_PER_PAIR)

    def start_out(c, outb, sem):
        dst = out_hbm.at[pl.ds(out_base + c * TOPS_PER_CHUNK, TOPS_PER_CHUNK)]
        pltpu.async_copy(outb, dst, sem)

    def wait_out(outb, sem):
        pltpu.make_async_copy(out_hbm.at[pl.ds(0, TOPS_PER_CHUNK)], outb,
                              sem).wait()

    bufs = (buf0, buf1, buf2, buf3, buf4, buf5, buf6, buf7)
    gsems = (sg0, sg1, sg2, sg3, sg4, sg5, sg6, sg7)
    outbs = (outb0, outb1)
    osems = (so0, so1)
    NBUF = 8
    LOOKAHEAD = 7
    for c in range(LOOKAHEAD):
        start_gather(c, bufs[c], gsems[c])

    def outer(i, _):
        c0 = NBUF * i
        for j in range(NBUF):
            c = c0 + j
            b = j
            ob = j % 2
            wait_gather(bufs[b], gsems[b])

            @pl.when(c >= 2)
            def _():
                wait_out(outbs[ob], osems[ob])

            accumulate(bufs[b], outbs[ob])
            start_out(c, outbs[ob], osems[ob])

            @pl.when(c + LOOKAHEAD < NCHUNK)
            def _():
                start_gather(c + LOOKAHEAD, bufs[(b + LOOKAHEAD) % NBUF],
                             gsems[(b + LOOKAHEAD) % NBUF])
        return 0

    lax.fori_loop(0, NCHUNK // NBUF, outer, 0)
    wait_out(outbs[0], osems[0])
    wait_out(outbs[1], osems[1])


@functools.partial(
    pl.kernel,
    out_type=jax.ShapeDtypeStruct((G_ROWS, D), jnp.float32),
    mesh=plsc.VectorSubcoreMesh(core_axis_name="c", subcore_axis_name="s"),
    scratch_types=[
        pltpu.VMEM((IDX_PER_W,), jnp.int32),
        pltpu.VMEM((CHUNK_ROWS, D), jnp.float32),
        pltpu.VMEM((CHUNK_ROWS, D), jnp.float32),
        pltpu.VMEM((CHUNK_ROWS, D), jnp.float32),
        pltpu.VMEM((CHUNK_ROWS, D), jnp.float32),
        pltpu.VMEM((CHUNK_ROWS, D), jnp.float32),
        pltpu.VMEM((CHUNK_ROWS, D), jnp.float32),
        pltpu.VMEM((CHUNK_ROWS, D), jnp.float32),
        pltpu.VMEM((CHUNK_ROWS, D), jnp.float32),
        pltpu.VMEM((TOPS_PER_CHUNK, D), jnp.float32),
        pltpu.VMEM((TOPS_PER_CHUNK, D), jnp.float32),
        pltpu.SemaphoreType.DMA,
        pltpu.SemaphoreType.DMA,
        pltpu.SemaphoreType.DMA,
        pltpu.SemaphoreType.DMA,
        pltpu.SemaphoreType.DMA,
        pltpu.SemaphoreType.DMA,
        pltpu.SemaphoreType.DMA,
        pltpu.SemaphoreType.DMA,
        pltpu.SemaphoreType.DMA,
        pltpu.SemaphoreType.DMA,
    ],
)
def _gather_segsum(table_hbm, idx_hbm, out_hbm, idx_v, buf0, buf1, buf2, buf3,
                   buf4, buf5, buf6, buf7, outb0, outb1, sg0, sg1, sg2, sg3,
                   sg4, sg5, sg6, sg7, so0, so1):
    _sc_gather_segsum(table_hbm, idx_hbm, out_hbm, idx_v, buf0, buf1, buf2,
                      buf3, buf4, buf5, buf6, buf7, outb0, outb1, sg0, sg1,
                      sg2, sg3, sg4, sg5, sg6, sg7, so0, so1)


ITEMS_PER_STEP = 8


def _tc_bilinear(g0_ref, w_ref, g1_ref, o_ref):
    g0 = g0_ref[...].astype(jnp.bfloat16)
    w = w_ref[...]
    h = jnp.dot(g0, w, preferred_element_type=jnp.float32)
    h = h.astype(jnp.bfloat16)
    scale = 1.0 / np.sqrt(D)
    for j in range(ITEMS_PER_STEP):
        rows = slice(j * NUM_TOP, (j + 1) * NUM_TOP)
        g1 = g1_ref[rows].astype(jnp.bfloat16)
        o_ref[j] = jnp.dot(h[rows], g1.T,
                           preferred_element_type=jnp.float32) * scale


def kernel(orig_fea, ind0, ind1, W):
    table = orig_fea.reshape(NPAIR * X, D)

    offs = jnp.arange(HK, dtype=jnp.int32) * BAG
    item = (jnp.arange(BS, dtype=jnp.int32) * (2 * X))[:, None, None]
    side = (jnp.arange(2, dtype=jnp.int32) * X)[:, None, None, None]
    idx_all = (jnp.stack([ind0, ind1]) + offs + item + side).reshape(-1)

    G = _gather_segsum(table, idx_all)
    W16 = W.astype(jnp.bfloat16)

    nstep = BS // ITEMS_PER_STEP
    blk = ITEMS_PER_STEP * NUM_TOP
    scores = pl.pallas_call(
        _tc_bilinear,
        grid=(nstep,),
        in_specs=[
            pl.BlockSpec((blk, D), lambda i: (i, 0)),
            pl.BlockSpec((D, D), lambda i: (0, 0)),
            pl.BlockSpec((blk, D), lambda i: (nstep + i, 0)),
        ],
        out_specs=pl.BlockSpec((ITEMS_PER_STEP, NUM_TOP, NUM_TOP),
                               lambda i: (i, 0, 0)),
        out_shape=jax.ShapeDtypeStruct((BS, NUM_TOP, NUM_TOP), jnp.float32),
    )(G, W16, G)

    scores = scores.reshape(BS, NUM_TOP * NUM_TOP, 1)

    ii, jj = jnp.meshgrid(jnp.arange(NUM_TOP, dtype=jnp.int32),
                          jnp.arange(NUM_TOP, dtype=jnp.int32), indexing="ij")
    pairs = jnp.stack([ii, jj], axis=-1)
    pairs = jnp.broadcast_to(pairs[None], (BS, NUM_TOP, NUM_TOP, 2))
    pairs = pairs.reshape(BS, NUM_TOP * NUM_TOP, 2)
    return scores, pairs


# speedup vs baseline: 1.0091x; 1.0091x over previous
import functools

import jax
import jax.numpy as jnp
import numpy as np
from jax import lax
from jax.experimental import pallas as pl
from jax.experimental.pallas import tpu as pltpu
from jax.experimental.pallas import tpu_sc as plsc

K_LEVEL = 16
HK = K_LEVEL // 2
MBS, NCOBJ = 4, 8
BS = MBS * NCOBJ
NUM_TOP = 128
BAG = 64
D = 512
X = BAG * HK
NPAIR = 2 * BS
ROWS_PER_PAIR = NUM_TOP
TOT_IDX = NPAIR * NUM_TOP * HK
G_ROWS = NPAIR * ROWS_PER_PAIR

NW = 32
PAIRS_PER_W = NPAIR // NW
IDX_PER_W = PAIRS_PER_W * NUM_TOP * HK
CHUNK_ROWS = 16
TOPS_PER_CHUNK = CHUNK_ROWS // HK
NCHUNK = IDX_PER_W // CHUNK_ROWS
LANES = 16
GROUPS = D // LANES


def _sc_gather_segsum(table_hbm, idx_hbm, out_hbm, idx_v, buf0, buf1, buf2,
                      buf3, buf4, buf5, buf6, buf7, outb0, outb1, sg0, sg1,
                      sg2, sg3, sg4, sg5, sg6, sg7, so0, so1):
    nc = 2
    wid = lax.axis_index("s") * nc + lax.axis_index("c")

    pltpu.sync_copy(idx_hbm.at[pl.ds(wid * IDX_PER_W, IDX_PER_W)], idx_v)

    def start_gather(c, buf, sem):
        src = table_hbm.at[idx_v.at[pl.ds(c * CHUNK_ROWS, CHUNK_ROWS)]]
        pltpu.async_copy(src, buf, sem)

    def wait_gather(buf, sem):
        pltpu.make_async_copy(table_hbm.at[pl.ds(0, CHUNK_ROWS)], buf,
                              sem).wait()

    def accumulate(buf, outb):
        @plsc.parallel_loop(0, TOPS_PER_CHUNK * GROUPS, unroll=4)
        def _(it):
            t = it // GROUPS
            g = it % GROUPS
            r0 = t * HK
            sl = pl.ds(g * LANES, LANES)
            v01 = buf[r0, sl] + buf[r0 + 1, sl]
            v23 = buf[r0 + 2, sl] + buf[r0 + 3, sl]
            v45 = buf[r0 + 4, sl] + buf[r0 + 5, sl]
            v67 = buf[r0 + 6, sl] + buf[r0 + 7, sl]
            outb[t, sl] = (v01 + v23) + (v45 + v67)

    out_base = wid * (PAIRS_PER_W * ROWS_PER_PAIR)

    def start_out(c, outb, sem):
        dst = out_hbm.at[pl.ds(out_base + c * TOPS_PER_CHUNK, TOPS_PER_CHUNK)]
        pltpu.async_copy(outb, dst, sem)

    def wait_out(outb, sem):
        pltpu.make_async_copy(out_hbm.at[pl.ds(0, TOPS_PER_CHUNK)], outb,
                              sem).wait()

    bufs = (buf0, buf1, buf2, buf3, buf4, buf5, buf6, buf7)
    gsems = (sg0, sg1, sg2, sg3, sg4, sg5, sg6, sg7)
    outbs = (outb0, outb1)
    osems = (so0, so1)
    NBUF = 8
    LOOKAHEAD = 7
    for c in range(LOOKAHEAD):
        start_gather(c, bufs[c], gsems[c])

    def outer(i, _):
        c0 = NBUF * i
        for j in range(NBUF):
            c = c0 + j
            b = j
            ob = j % 2
            wait_gather(bufs[b], gsems[b])

            @pl.when(c >= 2)
            def _():
                wait_out(outbs[ob], osems[ob])

            accumulate(bufs[b], outbs[ob])
            start_out(c, outbs[ob], osems[ob])

            @pl.when(c + LOOKAHEAD < NCHUNK)
            def _():
                start_gather(c + LOOKAHEAD, bufs[(b + LOOKAHEAD) % NBUF],
                             gsems[(b + LOOKAHEAD) % NBUF])
        return 0

    lax.fori_loop(0, NCHUNK // NBUF, outer, 0)
    wait_out(outbs[0], osems[0])
    wait_out(outbs[1], osems[1])


@functools.partial(
    pl.kernel,
    out_type=jax.ShapeDtypeStruct((G_ROWS, D), jnp.float32),
    mesh=plsc.VectorSubcoreMesh(core_axis_name="c", subcore_axis_name="s"),
    scratch_types=[
        pltpu.VMEM((IDX_PER_W,), jnp.int32),
        pltpu.VMEM((CHUNK_ROWS, D), jnp.float32),
        pltpu.VMEM((CHUNK_ROWS, D), jnp.float32),
        pltpu.VMEM((CHUNK_ROWS, D), jnp.float32),
        pltpu.VMEM((CHUNK_ROWS, D), jnp.float32),
        pltpu.VMEM((CHUNK_ROWS, D), jnp.float32),
        pltpu.VMEM((CHUNK_ROWS, D), jnp.float32),
        pltpu.VMEM((CHUNK_ROWS, D), jnp.float32),
        pltpu.VMEM((CHUNK_ROWS, D), jnp.float32),
        pltpu.VMEM((TOPS_PER_CHUNK, D), jnp.float32),
        pltpu.VMEM((TOPS_PER_CHUNK, D), jnp.float32),
        pltpu.SemaphoreType.DMA,
        pltpu.SemaphoreType.DMA,
        pltpu.SemaphoreType.DMA,
        pltpu.SemaphoreType.DMA,
        pltpu.SemaphoreType.DMA,
        pltpu.SemaphoreType.DMA,
        pltpu.SemaphoreType.DMA,
        pltpu.SemaphoreType.DMA,
        pltpu.SemaphoreType.DMA,
        pltpu.SemaphoreType.DMA,
    ],
)
def _gather_segsum(table_hbm, idx_hbm, out_hbm, idx_v, buf0, buf1, buf2, buf3,
                   buf4, buf5, buf6, buf7, outb0, outb1, sg0, sg1, sg2, sg3,
                   sg4, sg5, sg6, sg7, so0, so1):
    _sc_gather_segsum(table_hbm, idx_hbm, out_hbm, idx_v, buf0, buf1, buf2,
                      buf3, buf4, buf5, buf6, buf7, outb0, outb1, sg0, sg1,
                      sg2, sg3, sg4, sg5, sg6, sg7, so0, so1)


ITEMS_PER_STEP = 8


def _tc_bilinear(g0_ref, w_ref, g1_ref, o_ref):
    g0 = g0_ref[...].astype(jnp.bfloat16)
    w = w_ref[...]
    h = jnp.dot(g0, w, preferred_element_type=jnp.float32)
    h = h.astype(jnp.bfloat16)
    scale = 1.0 / np.sqrt(D)
    for j in range(ITEMS_PER_STEP):
        rows = slice(j * NUM_TOP, (j + 1) * NUM_TOP)
        g1 = g1_ref[rows].astype(jnp.bfloat16)
        o_ref[j] = jnp.dot(h[rows], g1.T,
                           preferred_element_type=jnp.float32) * scale


def kernel(orig_fea, ind0, ind1, W):
    table = orig_fea.reshape(NPAIR * X, D)

    offs = jnp.arange(HK, dtype=jnp.int32) * BAG
    item = (jnp.arange(BS, dtype=jnp.int32) * (2 * X))[:, None, None]
    side = (jnp.arange(2, dtype=jnp.int32) * X)[:, None, None, None]
    idx_all = (jnp.stack([ind0, ind1]) + offs + item + side).reshape(-1)

    G = _gather_segsum(table, idx_all)
    W16 = W.astype(jnp.bfloat16)

    nstep = BS // ITEMS_PER_STEP
    blk = ITEMS_PER_STEP * NUM_TOP
    scores = pl.pallas_call(
        _tc_bilinear,
        grid=(nstep,),
        in_specs=[
            pl.BlockSpec((blk, D), lambda i: (i, 0)),
            pl.BlockSpec((D, D), lambda i: (0, 0)),
            pl.BlockSpec((blk, D), lambda i: (nstep + i, 0)),
        ],
        out_specs=pl.BlockSpec((ITEMS_PER_STEP, NUM_TOP, NUM_TOP),
                               lambda i: (i, 0, 0)),
        out_shape=jax.ShapeDtypeStruct((BS, NUM_TOP, NUM_TOP), jnp.float32),
    )(G, W16, G)

    scores = scores.reshape(BS, NUM_TOP * NUM_TOP, 1)

    ii, jj = jnp.meshgrid(jnp.arange(NUM_TOP, dtype=jnp.int32),
                          jnp.arange(NUM_TOP, dtype=jnp.int32), indexing="ij")
    pairs = jnp.stack([ii, jj], axis=-1)
    pairs = jnp.broadcast_to(pairs[None], (BS, NUM_TOP, NUM_TOP, 2))
    pairs = pairs.reshape(BS, NUM_TOP * NUM_TOP, 2)
    return scores, pairs
